# baseline (speedup 1.0000x reference)
import functools

import jax
import jax.numpy as jnp
from jax import lax
from jax.experimental import pallas as pl
from jax.experimental.pallas import tpu as pltpu
from jax.experimental.pallas import tpu_sc as plsc

_M_EDGES = 10000
_RACC = 10240
_CPL = 128
_IB = 20
_STRIPE = _RACC // 16


def _sc_mesh():
    return plsc.VectorSubcoreMesh(core_axis_name="c", subcore_axis_name="s")


def _make_segsum(n_chunks):
    per_tile = n_chunks // 16

    @functools.partial(
        pl.kernel,
        out_type=jax.ShapeDtypeStruct((4, _RACC, 128), jnp.float32),
        mesh=_sc_mesh(),
        scratch_types=[
            pltpu.VMEM((_CPL,), jnp.int32),
            pltpu.VMEM((_CPL,), jnp.int32),
            pltpu.VMEM((_CPL,), jnp.int32),
            pltpu.VMEM((_CPL,), jnp.int32),
            pltpu.VMEM((_CPL, 128), jnp.float32),
            pltpu.VMEM((_CPL, 128), jnp.float32),
            pltpu.VMEM_SHARED((_RACC, 128), jnp.float32),
            pltpu.SemaphoreType.DMA,
            pltpu.SemaphoreType.DMA,
        ],
    )
    def seg(gidx_hbm, sidx_hbm, table_hbm, zeros_hbm, out_hbm,
            gv0, gv1, sv0, sv1, rows0, rows1, acc, sem0, sem1):
        c = lax.axis_index("c")
        s = lax.axis_index("s")
        base = s * per_tile
        gvs = (gv0, gv1)
        svs = (sv0, sv1)
        rowss = (rows0, rows1)
        sems = (sem0, sem1)
        for ci in range(2):
            @pl.when(c == ci)
            def _():
                for fi in range(2):
                    fc = 2 * ci + fi
                    for z in range(_STRIPE // 128):
                        pltpu.sync_copy(zeros_hbm,
                                        acc.at[pl.ds(s * _STRIPE + z * 128, 128)])
                    plsc.subcore_barrier()

                    def body(k, carry):
                        ch = base + k
                        pltpu.sync_copy(gidx_hbm.at[ch], gv0)
                        pltpu.sync_copy(sidx_hbm.at[ch], sv0)
                        pltpu.async_copy(
                            table_hbm.at[fc].at[gv0], rows0, sem0).wait()
                        pltpu.sync_copy(rows0, acc.at[sv0], add=True)
                        return carry

                    lax.fori_loop(0, per_tile, body, 0)
                    plsc.subcore_barrier()
                    pltpu.sync_copy(acc.at[pl.ds(s * _STRIPE, _STRIPE)],
                                    out_hbm.at[fc].at[pl.ds(s * _STRIPE, _STRIPE)])
                    plsc.subcore_barrier()

    return seg


def _make_degrees(n_chunks):
    per_tile = n_chunks // 16

    @functools.partial(
        pl.kernel,
        out_type=jax.ShapeDtypeStruct((2, _RACC), jnp.float32),
        mesh=_sc_mesh(),
        scratch_types=[
            pltpu.VMEM((_CPL,), jnp.int32),
            pltpu.VMEM((_CPL,), jnp.float32),
            pltpu.VMEM((_STRIPE,), jnp.float32),
            pltpu.VMEM_SHARED((_RACC,), jnp.float32),
            pltpu.SemaphoreType.DMA,
        ],
    )
    def deg(src_hbm, dst_hbm, ones_hbm, zeros_hbm, out_hbm, iv, ones_v, zv, acc, sem):
        c = lax.axis_index("c")
        s = lax.axis_index("s")
        pltpu.sync_copy(ones_hbm, ones_v)
        pltpu.sync_copy(zeros_hbm, zv)
        pltpu.sync_copy(zv, acc.at[pl.ds(s * _STRIPE, _STRIPE)])
        plsc.subcore_barrier()
        for ci in range(2):
            @pl.when(c == ci)
            def _():
                idx_hbm = src_hbm if ci == 0 else dst_hbm

                def body(k, carry):
                    pltpu.sync_copy(idx_hbm.at[s * per_tile + k], iv)
                    pltpu.sync_copy(ones_v, acc.at[iv], add=True)
                    return carry

                lax.fori_loop(0, per_tile, body, 0)
                plsc.subcore_barrier()
                pltpu.sync_copy(acc.at[pl.ds(s * _STRIPE, _STRIPE)],
                                out_hbm.at[ci].at[pl.ds(s * _STRIPE, _STRIPE)])

    return deg


def _mm_n2e(x, w):
    r_total, k = x.shape
    rb = 2000

    def body(x_ref, w_ref, o_ref):
        acc = jnp.dot(x_ref[...], w_ref[...], preferred_element_type=jnp.float32,
                      precision=lax.Precision.HIGHEST)
        for fo in range(4):
            o_ref[fo, :, :] = acc[:, fo * 128:(fo + 1) * 128]

    return pl.pallas_call(
        body,
        grid=(r_total // rb,),
        in_specs=[pl.BlockSpec((rb, k), lambda r: (r, 0)),
                  pl.BlockSpec((k, 512), lambda r: (0, 0))],
        out_specs=pl.BlockSpec((4, rb, 128), lambda r: (0, r, 0)),
        out_shape=jax.ShapeDtypeStruct((4, r_total, 128), jnp.float32),
    )(x, w)


def _mm_e2n(efull, wr):
    rb = 2000

    def body(e_ref, w_ref, o_ref):
        acc = jnp.zeros((rb, 512), jnp.float32)
        for f in range(4):
            acc = acc + jnp.dot(e_ref[f], w_ref[f], preferred_element_type=jnp.float32,
                                precision=lax.Precision.HIGHEST)
        for fo in range(4):
            o_ref[fo, :, :] = acc[:, fo * 128:(fo + 1) * 128]

    return pl.pallas_call(
        body,
        grid=(efull.shape[1] // rb,),
        in_specs=[pl.BlockSpec((4, rb, 128), lambda r: (0, r, 0)),
                  pl.BlockSpec((4, 128, 512), lambda r: (0, 0, 0))],
        out_specs=pl.BlockSpec((4, rb, 128), lambda r: (0, r, 0)),
        out_shape=jax.ShapeDtypeStruct((4, efull.shape[1], 128), jnp.float32),
    )(efull, wr)


def _ew_e(esum, xe, de, a):
    rb = 2000
    n_r = 10

    def body(es_ref, xe_ref, de_ref, a_ref, o_ref):
        r = pl.program_id(1)
        aa = a_ref[0]

        @pl.when(r < 5)
        def _():
            d = de_ref[...]
            scale = jnp.where(d > 0, 1.0 / d, 0.0)
            v = es_ref[0] * scale
            o_ref[0, :, :] = jnp.where(v >= 0, v, aa * v)

        @pl.when(r >= 5)
        def _():
            v = xe_ref[0]
            o_ref[0, :, :] = jnp.where(v >= 0, v, aa * v)

    return pl.pallas_call(
        body,
        grid=(4, n_r),
        in_specs=[
            pl.BlockSpec((1, rb, 128), lambda f, r: (f, jnp.where(r < 5, r, 0), 0)),
            pl.BlockSpec((1, rb, 128), lambda f, r: (f, jnp.where(r < 5, 0, r - 5), 0)),
            pl.BlockSpec((rb, 1), lambda f, r: (jnp.where(r < 5, r, 0), 0)),
            pl.BlockSpec(memory_space=pltpu.SMEM),
        ],
        out_specs=pl.BlockSpec((1, rb, 128), lambda f, r: (f, r, 0)),
        out_shape=jax.ShapeDtypeStruct((4, 20000, 128), jnp.float32),
    )(esum, xe, de, a)


def _ew_h(ns, xn, dn, a):
    rb = 2000

    def body(ns_ref, xn_ref, dn_ref, a_ref, o_ref):
        aa = a_ref[0]
        scale = 1.0 / (dn_ref[...] + 1.0)
        v = (ns_ref[0] + xn_ref[0]) * scale
        o_ref[...] = jnp.where(v >= 0, v, aa * v)

    return pl.pallas_call(
        body,
        grid=(4, 5),
        in_specs=[
            pl.BlockSpec((1, rb, 128), lambda f, r: (f, r, 0)),
            pl.BlockSpec((1, rb, 128), lambda f, r: (f, r + 5, 0)),
            pl.BlockSpec((rb, 1), lambda f, r: (r, 0)),
            pl.BlockSpec(memory_space=pltpu.SMEM),
        ],
        out_specs=pl.BlockSpec((rb, 128), lambda f, r: (r, f)),
        out_shape=jax.ShapeDtypeStruct((10000, 512), jnp.float32),
    )(ns, xn, dn, a)


def _ew_eout(efull):
    rb = 2000

    def body(e_ref, o_ref):
        o_ref[...] = e_ref[0]

    return pl.pallas_call(
        body,
        grid=(4, 5),
        in_specs=[pl.BlockSpec((1, rb, 128), lambda f, r: (f, r, 0))],
        out_specs=pl.BlockSpec((rb, 128), lambda f, r: (r, f)),
        out_shape=jax.ShapeDtypeStruct((10000, 512), jnp.float32),
    )(efull)


def kernel(x, hyperedge_index, num_nodes, num_edges,
           W_n2e1, W_e2n1, W_n2e2, W_e2n2, prelu_a):
    n_nodes = x.shape[0]
    nnz = hyperedge_index.shape[1]
    src = hyperedge_index[0].astype(jnp.int32)
    dst = hyperedge_index[1].astype(jnp.int32)

    n_chunks = -(-nnz // _CPL)
    n_chunks = -(-n_chunks // 16) * 16
    npad = n_chunks * _CPL - nnz
    pad_g = jnp.arange(npad, dtype=jnp.int32) % 128
    pad_s = _M_EDGES + jnp.arange(npad, dtype=jnp.int32) % 192
    src_g = jnp.concatenate([src, pad_g]).reshape(n_chunks, _CPL)
    dst_g = jnp.concatenate([dst, pad_g]).reshape(n_chunks, _CPL)
    src_s = jnp.concatenate([src, pad_s]).reshape(n_chunks, _CPL)
    dst_s = jnp.concatenate([dst, pad_s]).reshape(n_chunks, _CPL)

    zeros2d = jnp.zeros((_CPL, 128), jnp.float32)
    ones1 = jnp.ones((_CPL,), jnp.float32)
    zeros1 = jnp.zeros((_STRIPE,), jnp.float32)
    a_s = jnp.reshape(prelu_a, (1,))

    segsum = _make_segsum(n_chunks)
    deg = _make_degrees(n_chunks)(src_s, dst_s, ones1, zeros1)
    dn = deg[0, :n_nodes].reshape(n_nodes, 1)
    de = deg[1, :_M_EDGES].reshape(_M_EDGES, 1)

    wr1 = W_e2n1.reshape(4, 128, 512)
    wr2 = W_e2n2.reshape(4, 128, 512)

    xe1 = _mm_n2e(x, W_n2e1)
    es1 = segsum(src_g, dst_s, xe1, zeros2d)
    ef1 = _ew_e(es1, xe1, de, a_s)
    xn1 = _mm_e2n(ef1, wr1)
    ns1 = segsum(dst_g, src_s, xn1, zeros2d)
    h1 = _ew_h(ns1, xn1, dn, a_s)

    xe2 = _mm_n2e(h1, W_n2e2)
    es2 = segsum(src_g, dst_s, xe2, zeros2d)
    ef2 = _ew_e(es2, xe2, de, a_s)
    e_out = _ew_eout(ef2)
    xn2 = _mm_e2n(ef2, wr2)
    ns2 = segsum(dst_g, src_s, xn2, zeros2d)
    h2 = _ew_h(ns2, xn2, dn, a_s)

    return (h2, e_out)


# speedup vs baseline: 1.3053x; 1.3053x over previous
import functools

import jax
import jax.numpy as jnp
from jax import lax
from jax.experimental import pallas as pl
from jax.experimental.pallas import tpu as pltpu
from jax.experimental.pallas import tpu_sc as plsc

_M_EDGES = 10000
_RACC = 10240
_CPL = 128
_IB = 20
_STRIPE = _RACC // 16


def _sc_mesh():
    return plsc.VectorSubcoreMesh(core_axis_name="c", subcore_axis_name="s")


def _make_segsum(n_chunks):
    per_tile = n_chunks // 16

    @functools.partial(
        pl.kernel,
        out_type=jax.ShapeDtypeStruct((4, _RACC, 128), jnp.float32),
        mesh=_sc_mesh(),
        scratch_types=[
            pltpu.VMEM((_CPL,), jnp.int32),
            pltpu.VMEM((_CPL,), jnp.int32),
            pltpu.VMEM((_CPL,), jnp.int32),
            pltpu.VMEM((_CPL,), jnp.int32),
            pltpu.VMEM((_CPL, 128), jnp.float32),
            pltpu.VMEM((_CPL, 128), jnp.float32),
            pltpu.VMEM_SHARED((_RACC, 128), jnp.float32),
            pltpu.SemaphoreType.DMA,
            pltpu.SemaphoreType.DMA,
        ],
    )
    def seg(gidx_hbm, sidx_hbm, table_hbm, zeros_hbm, out_hbm,
            gv0, gv1, sv0, sv1, rows0, zrows, acc, sem0, sem1):
        c = lax.axis_index("c")
        s = lax.axis_index("s")
        base = s * per_tile
        gvs = (gv0, gv1)
        svs = (sv0, sv1)
        pltpu.sync_copy(zeros_hbm, zrows)
        for ci in range(2):
            @pl.when(c == ci)
            def _():
                for fi in range(2):
                    fc = 2 * ci + fi
                    for z in range(_STRIPE // 128):
                        pltpu.sync_copy(zrows,
                                        acc.at[pl.ds(s * _STRIPE + z * 128, 128)])
                    plsc.subcore_barrier()

                    def body(k, carry):
                        ch = base + k
                        pltpu.sync_copy(gidx_hbm.at[ch], gv0)
                        desc = pltpu.async_copy(
                            table_hbm.at[fc].at[gv0], rows0, sem0)
                        pltpu.sync_copy(sidx_hbm.at[ch], sv0)
                        desc.wait()
                        pltpu.sync_copy(rows0, acc.at[sv0], add=True)
                        return carry

                    lax.fori_loop(0, per_tile, body, 0)
                    plsc.subcore_barrier()
                    pltpu.sync_copy(acc.at[pl.ds(s * _STRIPE, _STRIPE)],
                                    out_hbm.at[fc].at[pl.ds(s * _STRIPE, _STRIPE)])
                    plsc.subcore_barrier()

    return seg


def _make_degrees(n_chunks):
    per_tile = n_chunks // 16

    @functools.partial(
        pl.kernel,
        out_type=jax.ShapeDtypeStruct((2, _RACC), jnp.float32),
        mesh=_sc_mesh(),
        scratch_types=[
            pltpu.VMEM((_CPL,), jnp.int32),
            pltpu.VMEM((_CPL,), jnp.float32),
            pltpu.VMEM((_STRIPE,), jnp.float32),
            pltpu.VMEM_SHARED((_RACC,), jnp.float32),
            pltpu.SemaphoreType.DMA,
        ],
    )
    def deg(src_hbm, dst_hbm, ones_hbm, zeros_hbm, out_hbm, iv, ones_v, zv, acc, sem):
        c = lax.axis_index("c")
        s = lax.axis_index("s")
        pltpu.sync_copy(ones_hbm, ones_v)
        pltpu.sync_copy(zeros_hbm, zv)
        pltpu.sync_copy(zv, acc.at[pl.ds(s * _STRIPE, _STRIPE)])
        plsc.subcore_barrier()
        for ci in range(2):
            @pl.when(c == ci)
            def _():
                idx_hbm = src_hbm if ci == 0 else dst_hbm

                def body(k, carry):
                    pltpu.sync_copy(idx_hbm.at[s * per_tile + k], iv)
                    pltpu.sync_copy(ones_v, acc.at[iv], add=True)
                    return carry

                lax.fori_loop(0, per_tile, body, 0)
                plsc.subcore_barrier()
                pltpu.sync_copy(acc.at[pl.ds(s * _STRIPE, _STRIPE)],
                                out_hbm.at[ci].at[pl.ds(s * _STRIPE, _STRIPE)])

    return deg


def _mm_n2e(x, w):
    r_total, k = x.shape
    rb = 2000

    def body(x_ref, w_ref, o_ref):
        acc = jnp.dot(x_ref[...], w_ref[...], preferred_element_type=jnp.float32,
                      precision=lax.Precision.DEFAULT)
        for fo in range(4):
            o_ref[fo, :, :] = acc[:, fo * 128:(fo + 1) * 128]

    return pl.pallas_call(
        body,
        grid=(r_total // rb,),
        in_specs=[pl.BlockSpec((rb, k), lambda r: (r, 0)),
                  pl.BlockSpec((k, 512), lambda r: (0, 0))],
        out_specs=pl.BlockSpec((4, rb, 128), lambda r: (0, r, 0)),
        out_shape=jax.ShapeDtypeStruct((4, r_total, 128), jnp.float32),
    )(x, w)


def _mm_e2n(efull, wr):
    rb = 2000

    def body(e_ref, w_ref, o_ref):
        acc = jnp.zeros((rb, 512), jnp.float32)
        for f in range(4):
            acc = acc + jnp.dot(e_ref[f], w_ref[f], preferred_element_type=jnp.float32,
                                precision=lax.Precision.DEFAULT)
        for fo in range(4):
            o_ref[fo, :, :] = acc[:, fo * 128:(fo + 1) * 128]

    return pl.pallas_call(
        body,
        grid=(efull.shape[1] // rb,),
        in_specs=[pl.BlockSpec((4, rb, 128), lambda r: (0, r, 0)),
                  pl.BlockSpec((4, 128, 512), lambda r: (0, 0, 0))],
        out_specs=pl.BlockSpec((4, rb, 128), lambda r: (0, r, 0)),
        out_shape=jax.ShapeDtypeStruct((4, efull.shape[1], 128), jnp.float32),
    )(efull, wr)


def _ew_e(esum, xe, de, a):
    rb = 2000
    n_r = 10

    def body(es_ref, xe_ref, de_ref, a_ref, o_ref):
        r = pl.program_id(1)
        aa = a_ref[0]

        @pl.when(r < 5)
        def _():
            d = de_ref[...]
            scale = jnp.where(d > 0, 1.0 / d, 0.0)
            v = es_ref[0] * scale
            o_ref[0, :, :] = jnp.where(v >= 0, v, aa * v)

        @pl.when(r >= 5)
        def _():
            v = xe_ref[0]
            o_ref[0, :, :] = jnp.where(v >= 0, v, aa * v)

    return pl.pallas_call(
        body,
        grid=(4, n_r),
        in_specs=[
            pl.BlockSpec((1, rb, 128), lambda f, r: (f, jnp.where(r < 5, r, 0), 0)),
            pl.BlockSpec((1, rb, 128), lambda f, r: (f, jnp.where(r < 5, 0, r - 5), 0)),
            pl.BlockSpec((rb, 1), lambda f, r: (jnp.where(r < 5, r, 0), 0)),
            pl.BlockSpec(memory_space=pltpu.SMEM),
        ],
        out_specs=pl.BlockSpec((1, rb, 128), lambda f, r: (f, r, 0)),
        out_shape=jax.ShapeDtypeStruct((4, 20000, 128), jnp.float32),
    )(esum, xe, de, a)


def _ew_h(ns, xn, dn, a):
    rb = 2000

    def body(ns_ref, xn_ref, dn_ref, a_ref, o_ref):
        aa = a_ref[0]
        scale = 1.0 / (dn_ref[...] + 1.0)
        v = (ns_ref[0] + xn_ref[0]) * scale
        o_ref[...] = jnp.where(v >= 0, v, aa * v)

    return pl.pallas_call(
        body,
        grid=(4, 5),
        in_specs=[
            pl.BlockSpec((1, rb, 128), lambda f, r: (f, r, 0)),
            pl.BlockSpec((1, rb, 128), lambda f, r: (f, r + 5, 0)),
            pl.BlockSpec((rb, 1), lambda f, r: (r, 0)),
            pl.BlockSpec(memory_space=pltpu.SMEM),
        ],
        out_specs=pl.BlockSpec((rb, 128), lambda f, r: (r, f)),
        out_shape=jax.ShapeDtypeStruct((10000, 512), jnp.float32),
    )(ns, xn, dn, a)


def _ew_eout(efull):
    rb = 2000

    def body(e_ref, o_ref):
        o_ref[...] = e_ref[0]

    return pl.pallas_call(
        body,
        grid=(4, 5),
        in_specs=[pl.BlockSpec((1, rb, 128), lambda f, r: (f, r, 0))],
        out_specs=pl.BlockSpec((rb, 128), lambda f, r: (r, f)),
        out_shape=jax.ShapeDtypeStruct((10000, 512), jnp.float32),
    )(efull)


def kernel(x, hyperedge_index, num_nodes, num_edges,
           W_n2e1, W_e2n1, W_n2e2, W_e2n2, prelu_a):
    n_nodes = x.shape[0]
    nnz = hyperedge_index.shape[1]
    src = hyperedge_index[0].astype(jnp.int32)
    dst = hyperedge_index[1].astype(jnp.int32)

    n_chunks = -(-nnz // _CPL)
    n_chunks = -(-n_chunks // 16) * 16
    npad = n_chunks * _CPL - nnz
    pad_g = jnp.arange(npad, dtype=jnp.int32) % 128
    pad_s = _M_EDGES + jnp.arange(npad, dtype=jnp.int32) % 192
    src_g = jnp.concatenate([src, pad_g]).reshape(n_chunks, _CPL)
    dst_g = jnp.concatenate([dst, pad_g]).reshape(n_chunks, _CPL)
    src_s = jnp.concatenate([src, pad_s]).reshape(n_chunks, _CPL)
    dst_s = jnp.concatenate([dst, pad_s]).reshape(n_chunks, _CPL)

    zeros2d = jnp.zeros((_CPL, 128), jnp.float32)
    ones1 = jnp.ones((_CPL,), jnp.float32)
    zeros1 = jnp.zeros((_STRIPE,), jnp.float32)
    a_s = jnp.reshape(prelu_a, (1,))

    segsum = _make_segsum(n_chunks)
    deg = _make_degrees(n_chunks)(src_s, dst_s, ones1, zeros1)
    dn = deg[0, :n_nodes].reshape(n_nodes, 1)
    de = deg[1, :_M_EDGES].reshape(_M_EDGES, 1)

    wr1 = W_e2n1.reshape(4, 128, 512)
    wr2 = W_e2n2.reshape(4, 128, 512)

    xe1 = _mm_n2e(x, W_n2e1)
    es1 = segsum(src_g, dst_s, xe1, zeros2d)
    ef1 = _ew_e(es1, xe1, de, a_s)
    xn1 = _mm_e2n(ef1, wr1)
    ns1 = segsum(dst_g, src_s, xn1, zeros2d)
    h1 = _ew_h(ns1, xn1, dn, a_s)

    xe2 = _mm_n2e(h1, W_n2e2)
    es2 = segsum(src_g, dst_s, xe2, zeros2d)
    ef2 = _ew_e(es2, xe2, de, a_s)
    e_out = _ew_eout(ef2)
    xn2 = _mm_e2n(ef2, wr2)
    ns2 = segsum(dst_g, src_s, xn2, zeros2d)
    h2 = _ew_h(ns2, xn2, dn, a_s)

    return (h2, e_out)


# speedup vs baseline: 1.3089x; 1.0027x over previous
import functools

import jax
import jax.numpy as jnp
from jax import lax
from jax.experimental import pallas as pl
from jax.experimental.pallas import tpu as pltpu
from jax.experimental.pallas import tpu_sc as plsc

_M_EDGES = 10000
_RACC = 10240
_CPL = 128
_IB = 20
_STRIPE = _RACC // 16


def _sc_mesh():
    return plsc.VectorSubcoreMesh(core_axis_name="c", subcore_axis_name="s")


def _make_segsum(n_chunks):
    per_tile = n_chunks // 16

    @functools.partial(
        pl.kernel,
        out_type=jax.ShapeDtypeStruct((4, _RACC, 128), jnp.float32),
        mesh=_sc_mesh(),
        scratch_types=[
            pltpu.VMEM((_CPL,), jnp.int32),
            pltpu.VMEM((_CPL,), jnp.int32),
            pltpu.VMEM((_CPL, 128), jnp.float32),
            pltpu.VMEM((_CPL, 128), jnp.float32),
            pltpu.VMEM_SHARED((_RACC, 128), jnp.float32),
            pltpu.SemaphoreType.DMA,
        ],
    )
    def seg(gidx_hbm, sidx_hbm, table_hbm, zeros_hbm, out_hbm,
            gv, sv, rows, zrows, acc, sem):
        c = lax.axis_index("c")
        s = lax.axis_index("s")
        base = s * per_tile
        pltpu.sync_copy(zeros_hbm, zrows)
        for ci in range(2):
            @pl.when(c == ci)
            def _():
                for fi in range(2):
                    fc = 2 * ci + fi
                    for z in range(_STRIPE // 128):
                        pltpu.sync_copy(zrows,
                                        acc.at[pl.ds(s * _STRIPE + z * 128, 128)])
                    plsc.subcore_barrier()

                    def body(k, carry):
                        ch = base + k
                        pltpu.sync_copy(gidx_hbm.at[ch], gv)
                        desc = pltpu.async_copy(table_hbm.at[fc].at[gv], rows, sem)
                        pltpu.sync_copy(sidx_hbm.at[ch], sv)
                        desc.wait()
                        pltpu.sync_copy(rows, acc.at[sv], add=True)
                        return carry

                    lax.fori_loop(0, per_tile, body, 0)
                    plsc.subcore_barrier()
                    pltpu.sync_copy(acc.at[pl.ds(s * _STRIPE, _STRIPE)],
                                    out_hbm.at[fc].at[pl.ds(s * _STRIPE, _STRIPE)])
                    plsc.subcore_barrier()

    return seg


def _make_degrees(n_chunks):
    per_tile = n_chunks // 16

    @functools.partial(
        pl.kernel,
        out_type=jax.ShapeDtypeStruct((2, _RACC), jnp.float32),
        mesh=_sc_mesh(),
        scratch_types=[
            pltpu.VMEM((_CPL,), jnp.int32),
            pltpu.VMEM((_CPL,), jnp.float32),
            pltpu.VMEM((_STRIPE,), jnp.float32),
            pltpu.VMEM_SHARED((_RACC,), jnp.float32),
            pltpu.SemaphoreType.DMA,
        ],
    )
    def deg(src_hbm, dst_hbm, ones_hbm, zeros_hbm, out_hbm, iv, ones_v, zv, acc, sem):
        c = lax.axis_index("c")
        s = lax.axis_index("s")
        pltpu.sync_copy(ones_hbm, ones_v)
        pltpu.sync_copy(zeros_hbm, zv)
        pltpu.sync_copy(zv, acc.at[pl.ds(s * _STRIPE, _STRIPE)])
        plsc.subcore_barrier()
        for ci in range(2):
            @pl.when(c == ci)
            def _():
                idx_hbm = src_hbm if ci == 0 else dst_hbm

                def body(k, carry):
                    pltpu.sync_copy(idx_hbm.at[s * per_tile + k], iv)
                    pltpu.sync_copy(ones_v, acc.at[iv], add=True)
                    return carry

                lax.fori_loop(0, per_tile, body, 0)
                plsc.subcore_barrier()
                pltpu.sync_copy(acc.at[pl.ds(s * _STRIPE, _STRIPE)],
                                out_hbm.at[ci].at[pl.ds(s * _STRIPE, _STRIPE)])

    return deg


def _mm_n2e(x, w):
    r_total, k = x.shape
    rb = 2000

    def body(x_ref, w_ref, o_ref):
        acc = jnp.dot(x_ref[...], w_ref[...], preferred_element_type=jnp.float32,
                      precision=lax.Precision.DEFAULT)
        for fo in range(4):
            o_ref[fo, :, :] = acc[:, fo * 128:(fo + 1) * 128]

    return pl.pallas_call(
        body,
        grid=(r_total // rb,),
        in_specs=[pl.BlockSpec((rb, k), lambda r: (r, 0)),
                  pl.BlockSpec((k, 512), lambda r: (0, 0))],
        out_specs=pl.BlockSpec((4, rb, 128), lambda r: (0, r, 0)),
        out_shape=jax.ShapeDtypeStruct((4, r_total, 128), jnp.float32),
    )(x, w)


def _mm_e2n(efull, wr):
    rb = 2000

    def body(e_ref, w_ref, o_ref):
        acc = jnp.zeros((rb, 512), jnp.float32)
        for f in range(4):
            acc = acc + jnp.dot(e_ref[f], w_ref[f], preferred_element_type=jnp.float32,
                                precision=lax.Precision.DEFAULT)
        for fo in range(4):
            o_ref[fo, :, :] = acc[:, fo * 128:(fo + 1) * 128]

    return pl.pallas_call(
        body,
        grid=(efull.shape[1] // rb,),
        in_specs=[pl.BlockSpec((4, rb, 128), lambda r: (0, r, 0)),
                  pl.BlockSpec((4, 128, 512), lambda r: (0, 0, 0))],
        out_specs=pl.BlockSpec((4, rb, 128), lambda r: (0, r, 0)),
        out_shape=jax.ShapeDtypeStruct((4, efull.shape[1], 128), jnp.float32),
    )(efull, wr)


def _ew_e(esum, xe, de, a):
    rb = 2000
    n_r = 10

    def body(es_ref, xe_ref, de_ref, a_ref, o_ref):
        r = pl.program_id(1)
        aa = a_ref[0]

        @pl.when(r < 5)
        def _():
            d = de_ref[...]
            scale = jnp.where(d > 0, 1.0 / d, 0.0)
            v = es_ref[0] * scale
            o_ref[0, :, :] = jnp.where(v >= 0, v, aa * v)

        @pl.when(r >= 5)
        def _():
            v = xe_ref[0]
            o_ref[0, :, :] = jnp.where(v >= 0, v, aa * v)

    return pl.pallas_call(
        body,
        grid=(4, n_r),
        in_specs=[
            pl.BlockSpec((1, rb, 128), lambda f, r: (f, jnp.where(r < 5, r, 0), 0)),
            pl.BlockSpec((1, rb, 128), lambda f, r: (f, jnp.where(r < 5, 0, r - 5), 0)),
            pl.BlockSpec((rb, 1), lambda f, r: (jnp.where(r < 5, r, 0), 0)),
            pl.BlockSpec(memory_space=pltpu.SMEM),
        ],
        out_specs=pl.BlockSpec((1, rb, 128), lambda f, r: (f, r, 0)),
        out_shape=jax.ShapeDtypeStruct((4, 20000, 128), jnp.float32),
    )(esum, xe, de, a)


def _ew_h(ns, xn, dn, a):
    rb = 2000

    def body(ns_ref, xn_ref, dn_ref, a_ref, o_ref):
        aa = a_ref[0]
        scale = 1.0 / (dn_ref[...] + 1.0)
        v = (ns_ref[0] + xn_ref[0]) * scale
        o_ref[...] = jnp.where(v >= 0, v, aa * v)

    return pl.pallas_call(
        body,
        grid=(4, 5),
        in_specs=[
            pl.BlockSpec((1, rb, 128), lambda f, r: (f, r, 0)),
            pl.BlockSpec((1, rb, 128), lambda f, r: (f, r + 5, 0)),
            pl.BlockSpec((rb, 1), lambda f, r: (r, 0)),
            pl.BlockSpec(memory_space=pltpu.SMEM),
        ],
        out_specs=pl.BlockSpec((rb, 128), lambda f, r: (r, f)),
        out_shape=jax.ShapeDtypeStruct((10000, 512), jnp.float32),
    )(ns, xn, dn, a)


def _ew_eout(efull):
    rb = 2000

    def body(e_ref, o_ref):
        o_ref[...] = e_ref[0]

    return pl.pallas_call(
        body,
        grid=(4, 5),
        in_specs=[pl.BlockSpec((1, rb, 128), lambda f, r: (f, r, 0))],
        out_specs=pl.BlockSpec((rb, 128), lambda f, r: (r, f)),
        out_shape=jax.ShapeDtypeStruct((10000, 512), jnp.float32),
    )(efull)


def kernel(x, hyperedge_index, num_nodes, num_edges,
           W_n2e1, W_e2n1, W_n2e2, W_e2n2, prelu_a):
    n_nodes = x.shape[0]
    nnz = hyperedge_index.shape[1]
    src = hyperedge_index[0].astype(jnp.int32)
    dst = hyperedge_index[1].astype(jnp.int32)

    n_chunks = -(-nnz // _CPL)
    n_chunks = -(-n_chunks // 16) * 16
    npad = n_chunks * _CPL - nnz
    pad_g = jnp.arange(npad, dtype=jnp.int32) % 128
    pad_s = _M_EDGES + jnp.arange(npad, dtype=jnp.int32) % 192
    src_g = jnp.concatenate([src, pad_g]).reshape(n_chunks, _CPL)
    dst_g = jnp.concatenate([dst, pad_g]).reshape(n_chunks, _CPL)
    src_s = jnp.concatenate([src, pad_s]).reshape(n_chunks, _CPL)
    dst_s = jnp.concatenate([dst, pad_s]).reshape(n_chunks, _CPL)

    zeros2d = jnp.zeros((_CPL, 128), jnp.float32)
    ones1 = jnp.ones((_CPL,), jnp.float32)
    zeros1 = jnp.zeros((_STRIPE,), jnp.float32)
    a_s = jnp.reshape(prelu_a, (1,))

    segsum = _make_segsum(n_chunks)
    deg = _make_degrees(n_chunks)(src_s, dst_s, ones1, zeros1)
    dn = deg[0, :n_nodes].reshape(n_nodes, 1)
    de = deg[1, :_M_EDGES].reshape(_M_EDGES, 1)

    wr1 = W_e2n1.reshape(4, 128, 512)
    wr2 = W_e2n2.reshape(4, 128, 512)

    xe1 = _mm_n2e(x, W_n2e1)
    es1 = segsum(src_g, dst_s, xe1, zeros2d)
    ef1 = _ew_e(es1, xe1, de, a_s)
    xn1 = _mm_e2n(ef1, wr1)
    ns1 = segsum(dst_g, src_s, xn1, zeros2d)
    h1 = _ew_h(ns1, xn1, dn, a_s)

    xe2 = _mm_n2e(h1, W_n2e2)
    es2 = segsum(src_g, dst_s, xe2, zeros2d)
    ef2 = _ew_e(es2, xe2, de, a_s)
    e_out = _ew_eout(ef2)
    xn2 = _mm_e2n(ef2, wr2)
    ns2 = segsum(dst_g, src_s, xn2, zeros2d)
    h2 = _ew_h(ns2, xn2, dn, a_s)

    return (h2, e_out)
